# baseline (speedup 1.0000x reference)
import functools

import jax
import jax.numpy as jnp
from jax import lax
from jax.experimental import pallas as pl
from jax.experimental.pallas import tpu as pltpu
from jax.experimental.pallas import tpu_sc as plsc

_CH = 128
_LANES = 128


def _sc_info():
    try:
        info = plsc.get_sparse_core_info()
        return info.num_cores, info.num_subcores
    except Exception:
        return 2, 16


def _sc_gather(nidx, tidx, uidx, entity_table, user_table, *, nc, ns):
    nw = nc * ns
    n_chunks = nidx.shape[0]
    b_chunks = tidx.shape[0]
    d = entity_table.shape[1]
    assert n_chunks % nw == 0 and b_chunks % nw == 0
    ncw = n_chunks // nw
    bcw = b_chunks // nw

    mesh = plsc.VectorSubcoreMesh(
        core_axis_name="c", subcore_axis_name="s", num_cores=nc,
        num_subcores=ns)

    BK = n_chunks * _CH
    B = b_chunks * _CH

    @functools.partial(
        pl.kernel,
        out_type=[
            jax.ShapeDtypeStruct((BK, d), jnp.float32),
            jax.ShapeDtypeStruct((B, d), jnp.float32),
            jax.ShapeDtypeStruct((B, d), jnp.float32),
        ],
        mesh=mesh,
        compiler_params=pltpu.CompilerParams(use_tc_tiling_on_sc=False),
        scratch_types=[
            pltpu.VMEM((ncw, _CH), jnp.int32),
            pltpu.VMEM((bcw, _CH), jnp.int32),
            pltpu.VMEM((bcw, _CH), jnp.int32),
            pltpu.VMEM((_CH, d), jnp.float32),
            pltpu.SemaphoreType.DMA,
        ],
    )
    def k(nidx_hbm, tidx_hbm, uidx_hbm, etab, utab, out_n, out_t, out_u,
          nidx_v, tidx_v, uidx_v, rows_v, sem):
        wid = lax.axis_index("s") * nc + lax.axis_index("c")

        pltpu.sync_copy(nidx_hbm.at[pl.ds(wid * ncw, ncw)], nidx_v)
        pltpu.sync_copy(tidx_hbm.at[pl.ds(wid * bcw, bcw)], tidx_v)
        pltpu.sync_copy(uidx_hbm.at[pl.ds(wid * bcw, bcw)], uidx_v)

        def run(idx_v, table, out, cpw, _):
            base = wid * cpw * _CH

            def body(j, carry):
                pltpu.async_copy(table.at[idx_v.at[j]], rows_v, sem).wait()
                pltpu.sync_copy(rows_v,
                                out.at[pl.ds(base + j * _CH, _CH)])
                return carry

            lax.fori_loop(0, cpw, body, 0, unroll=False)

        run(nidx_v, etab, out_n, ncw, 0)
        run(tidx_v, etab, out_t, bcw, 1)
        run(uidx_v, utab, out_u, bcw, 2)

    return k(nidx, tidx, uidx, entity_table, user_table)


def _tc_attention_body(k_neigh, d, rows, tgt_ref, neigh_ref, usr_ref,
                       mseg_ref, a2m_ref, mrow_ref, wk_ref, wa1_ref,
                       hsd_ref, out_ref):
    nf = neigh_ref[...]
    t = tgt_ref[...]
    u = usr_ref[...]

    sst = jnp.sum(t * t, axis=1, keepdims=True)
    ts = (t @ wa1_ref[...]) * jnp.minimum(lax.rsqrt(sst), 1.0)

    ssq = (nf * nf) @ mseg_ref[...]
    inv = jnp.minimum(lax.rsqrt(ssq), 1.0)
    dot = nf @ a2m_ref[...]
    e = ts + dot * inv
    e = jnp.where(e > 0, e, 0.2 * e)
    mx = jnp.max(e, axis=1, keepdims=True)
    p = jnp.exp(e - mx)
    coef = (p / jnp.sum(p, axis=1, keepdims=True)) * inv

    cexp = coef @ mrow_ref[...]
    head = (nf * cexp) @ wk_ref[...]

    ssu = jnp.sum(u * u, axis=1, keepdims=True)
    uhs = (u @ hsd_ref[...]) * jnp.minimum(lax.rsqrt(ssu), 1.0)

    uv = jnp.sum(head * uhs, axis=1, keepdims=True)
    out_ref[...] = 1.0 / (1.0 + jnp.exp(-uv))


def _tc_attention(tgt, neigh, usr, w, a, *, b, k_neigh, d, rows):
    h = w.shape[1]
    kd = k_neigh * d
    f32 = jnp.float32

    wa1 = w @ a[:h]
    wa2 = (w @ a[h:])[:, 0]
    seg = (jnp.arange(kd)[:, None] // d
           == jnp.arange(k_neigh)[None, :]).astype(f32)
    a2m = seg * jnp.tile(wa2, k_neigh)[:, None]
    mrow = seg.T
    wk = jnp.tile(w, (k_neigh, 1))
    hsd = (jnp.arange(d)[:, None] % h
           == jnp.arange(h)[None, :]).astype(f32)

    grid = b // rows
    out = pl.pallas_call(
        functools.partial(_tc_attention_body, k_neigh, d, rows),
        grid=(grid,),
        in_specs=[
            pl.BlockSpec((rows, d), lambda g: (g, 0)),
            pl.BlockSpec((rows, kd), lambda g: (g, 0)),
            pl.BlockSpec((rows, d), lambda g: (g, 0)),
            pl.BlockSpec((kd, k_neigh), lambda g: (0, 0)),
            pl.BlockSpec((kd, k_neigh), lambda g: (0, 0)),
            pl.BlockSpec((k_neigh, kd), lambda g: (0, 0)),
            pl.BlockSpec((kd, h), lambda g: (0, 0)),
            pl.BlockSpec((d, 1), lambda g: (0, 0)),
            pl.BlockSpec((d, h), lambda g: (0, 0)),
        ],
        out_specs=pl.BlockSpec((rows, 1), lambda g: (g, 0)),
        out_shape=jax.ShapeDtypeStruct((b, 1), jnp.float32),
    )(tgt, neigh, usr, seg, a2m, mrow, wk, wa1, hsd)
    return out.reshape(b)


def kernel(u, i, neighbors, entity_table, user_table, W, a):
    b, k_neigh = neighbors.shape
    d = entity_table.shape[1]
    nc, ns = _sc_info()

    nidx = neighbors.reshape(b * k_neigh // _CH, _CH).astype(jnp.int32)
    tidx = i.reshape(b // _CH, _CH).astype(jnp.int32)
    uidx = u.reshape(b // _CH, _CH).astype(jnp.int32)

    neigh_g, tgt_g, usr_g = _sc_gather(
        nidx, tidx, uidx, entity_table, user_table, nc=nc, ns=ns)
    neigh_f = neigh_g.reshape(b, k_neigh * d)
    return _tc_attention(tgt_g, neigh_f, usr_g, W, a,
                         b=b, k_neigh=k_neigh, d=d, rows=1024)
